# Initial kernel scaffold; baseline (speedup 1.0000x reference)
import jax
import jax.numpy as jnp
from jax.experimental import pallas as pl


def kernel(boxes, scores):
    raise NotImplementedError("write your pallas kernel here")


# speedup vs baseline: 8.5313x; 8.5313x over previous
import jax
import jax.numpy as jnp
from jax import lax
from jax.experimental import pallas as pl
from jax.experimental.pallas import tpu as pltpu

_N = 20000
_TOPK = 400
_DET = 200
_SCORE_THRESH = 0.01
_NMS_THRESH = 0.5

_ROWS = 160
_LANES = 128
_NPAD = _ROWS * _LANES

_HI = jax.lax.Precision.HIGHEST
_f32 = jnp.float32
_i32 = jnp.int32


def _dot(a, b, ca, cb):
    return lax.dot_general(a, b, (((ca,), (cb,)), ((), ())),
                           precision=_HI, preferred_element_type=_f32)


def _nms_kernel(s_ref, x1_ref, y1_ref, x2_ref, y2_ref, out_ref, sup_ref):
    s = s_ref[:]
    key = jnp.where(s > _SCORE_THRESH,
                    lax.bitcast_convert_type(s, _i32), 0)

    g0 = jnp.sum(jnp.where(key > 0, 1, 0))

    def bs_body(i, u):
        t = u | (1 << (30 - i))
        cnt = jnp.sum(jnp.where(key > t, 1, 0))
        return jnp.where(cnt >= _TOPK, t, u)

    u = lax.fori_loop(0, 31, bs_body, jnp.int32(0))
    T = jnp.where(g0 >= _TOPK, u + 1, 0)
    t_is_pos = (T > 0).astype(_f32)
    t_score = jnp.where(T > 0, lax.bitcast_convert_type(T, _f32), 0.0)

    mgt = (key > T).astype(_f32)
    meq = (key == T).astype(_f32)
    n1 = jnp.sum(mgt)

    s_clean = jnp.where(key > 0, s, 0.0)

    kA = lax.broadcasted_iota(_i32, (_LANES, _LANES), 0)
    kB = lax.broadcasted_iota(_i32, (_LANES, _LANES), 1)
    tri_le = (kA <= kB).astype(_f32)
    rA = lax.broadcasted_iota(_i32, (_ROWS, _ROWS), 0)
    rB = lax.broadcasted_iota(_i32, (_ROWS, _ROWS), 1)
    tri_lt_r = (rB < rA).astype(_f32)
    ones_l = jnp.ones((_LANES, 1), _f32)
    ident_r = (rA == rB).astype(_f32)

    lane160 = lax.broadcasted_iota(_i32, (1, _ROWS), 1)
    lane128k = lax.broadcasted_iota(_i32, (_TOPK, _LANES), 1)
    slot = lax.broadcasted_iota(_i32, (_TOPK, 1), 0)
    slot_f = slot.astype(_f32)

    def class_gather(mask, planes, t_target):
        rowcum = _dot(mask, tri_le, 1, 0)
        rowtot = _dot(mask, ones_l, 1, 0)
        prefix = _dot(tri_lt_r, rowtot, 1, 0)
        cum = rowcum + prefix
        prefix_row = _dot(prefix, ident_r, 0, 0)
        rs = jnp.sum(jnp.where(prefix_row < t_target, 1.0, 0.0),
                     axis=1, keepdims=True).astype(_i32) - 1
        rs = jnp.maximum(rs, 0)
        RS = (lane160 == rs).astype(_f32)
        bcum = _dot(RS, cum, 1, 0)
        bmask = _dot(RS, mask, 1, 0)
        csm = jnp.where((bcum == t_target) & (bmask > 0.5), 1.0, 0.0)
        outs = []
        for p in planes:
            bp = _dot(RS, p, 1, 0)
            outs.append(jnp.sum(bp * csm, axis=1, keepdims=True))
        hit = jnp.sum(csm, axis=1, keepdims=True)
        return outs, hit

    x1p = x1_ref[:]
    y1p = y1_ref[:]
    x2p = x2_ref[:]
    y2p = y2_ref[:]

    gt_t = slot_f + 1.0
    (gx1, gy1, gx2, gy2, gsc), _ = class_gather(
        mgt, [x1p, y1p, x2p, y2p, s_clean], gt_t)
    eq_t = slot_f + 1.0 - n1
    (ex1, ey1, ex2, ey2), eq_hit = class_gather(
        meq, [x1p, y1p, x2p, y2p], eq_t)

    x1c = gx1 + ex1
    y1c = gy1 + ey1
    x2c = gx2 + ex2
    y2c = gy2 + ey2
    scc = gsc + eq_hit * t_score

    kTA = lax.broadcasted_iota(_i32, (_TOPK, _TOPK), 0)
    kTB = lax.broadcasted_iota(_i32, (_TOPK, _TOPK), 1)
    ident_k = (kTA == kTB).astype(_f32)

    def t_row(col):
        return _dot(col, ident_k, 0, 0)

    s_cmp_c = jnp.where(scc > 0.0, scc, -jnp.inf)
    scr = t_row(scc)
    s_cmp_r = jnp.where(scr > 0.0, scr, -jnp.inf)
    cmat = jnp.where((s_cmp_r > s_cmp_c)
                     | ((s_cmp_r == s_cmp_c) & (kTB < kTA)), 1.0, 0.0)
    rank = jnp.sum(cmat, axis=1, keepdims=True)
    rank_row = t_row(rank)
    P = jnp.where(kTA.astype(_f32) == jnp.broadcast_to(rank_row,
                                                       (_TOPK, _TOPK)),
                  1.0, 0.0)
    cand = jnp.concatenate([x1c, y1c, x2c, y2c, scc], axis=1)
    srt = _dot(P, cand, 1, 0)
    x1s = srt[:, 0:1]
    y1s = srt[:, 1:2]
    x2s = srt[:, 2:3]
    y2s = srt[:, 3:4]
    sss = srt[:, 4:5]
    x1r = t_row(x1s)
    y1r = t_row(y1s)
    x2r = t_row(x2s)
    y2r = t_row(y2s)
    ssr = t_row(sss)

    xx1 = jnp.maximum(x1s, x1r)
    yy1 = jnp.maximum(y1s, y1r)
    xx2 = jnp.minimum(x2s, x2r)
    yy2 = jnp.minimum(y2s, y2r)
    iw = jnp.maximum(xx2 - xx1, 0.0)
    ih = jnp.maximum(yy2 - yy1, 0.0)
    inter = iw * ih
    area_i = (x2s - x1s) * (y2s - y1s)
    area_j = (x2r - x1r) * (y2r - y1r)
    iou = inter / (area_i + area_j - inter + 1e-9)
    sup_ref[:] = jnp.where((iou > _NMS_THRESH) & (kTB > kTA), 1.0, 0.0)

    laneK = lax.broadcasted_iota(_i32, (1, _TOPK), 1)

    def nms_body(i, keep):
        ki = jnp.sum(jnp.where(laneK == i, keep, 0.0))
        row = sup_ref[pl.ds(i, 1), :]
        return keep * (1.0 - row * ki)

    keep = lax.fori_loop(0, _TOPK, nms_body, jnp.ones((1, _TOPK), _f32))

    kv = keep * jnp.where(ssr > 0.0, 1.0, 0.0)
    tri_k = (kTA <= kTB).astype(_f32)
    pos = _dot(kv, tri_k, 1, 0)
    p = pos.astype(_i32) - 1
    rows200 = lax.broadcasted_iota(_i32, (_DET, _TOPK), 0)
    onehot = jnp.where((rows200 == jnp.broadcast_to(p, (_DET, _TOPK)))
                       & (jnp.broadcast_to(kv, (_DET, _TOPK)) > 0.5),
                       1.0, 0.0)
    data = jnp.concatenate([x1s, y1s, x2s, y2s, sss], axis=1)
    out_ref[:] = _dot(onehot, data, 1, 0)


@jax.jit
def kernel(boxes, scores):
    pad = jnp.full((_NPAD - _N,), -1.0, _f32)
    s2d = jnp.concatenate([scores, pad], axis=0).reshape(_ROWS, _LANES)
    bpad = jnp.zeros((_NPAD - _N, 4), _f32)
    b2 = jnp.concatenate([boxes, bpad], axis=0)
    x1 = b2[:, 0].reshape(_ROWS, _LANES)
    y1 = b2[:, 1].reshape(_ROWS, _LANES)
    x2 = b2[:, 2].reshape(_ROWS, _LANES)
    y2 = b2[:, 3].reshape(_ROWS, _LANES)

    return pl.pallas_call(
        _nms_kernel,
        out_shape=jax.ShapeDtypeStruct((_DET, 5), _f32),
        scratch_shapes=[
            pltpu.VMEM((_TOPK, _TOPK), _f32),
        ],
    )(s2d, x1, y1, x2, y2)
